# Initial kernel scaffold; baseline (speedup 1.0000x reference)
import jax
import jax.numpy as jnp
from jax.experimental import pallas as pl


def kernel(boxes, scores):
    raise NotImplementedError("write your pallas kernel here")


# speedup vs baseline: 1.2067x; 1.2067x over previous
import jax
import jax.numpy as jnp
from jax.experimental import pallas as pl

_N_RAW = 20000
_NP = 20480
_TILE = 256
_CH = 512
_K = 1000
_KP = 1024
_BLK = 128
_NBLK = _KP // _BLK
_THR = 0.7
_SCENE = 128.0


def _fiota(shape, dim):
    return jax.lax.broadcasted_iota(jnp.int32, shape, dim).astype(jnp.float32)


def _rank_select_kernel(s_col_ref, s_row_ref, aug_ref, out_ref):
    pid = pl.program_id(0)
    nprog = pl.num_programs(0)
    s_col = s_col_ref[...]
    i_glob = (pid * _TILE
              + _fiota( (_TILE, 1), 0))

    def body(c, rank):
        off = c * _CH
        s_chunk = s_row_ref[:, pl.ds(off, _CH)]
        j_glob = (off.astype(jnp.float32)
                  + _fiota( (1, _CH), 1))
        gt = s_chunk > s_col
        tie = (s_chunk == s_col) & (j_glob < i_glob)
        cnt = jnp.where(gt | tie, 1.0, 0.0)
        return rank + jnp.sum(cnt, axis=1, keepdims=True)

    rank = jax.lax.fori_loop(0, _NP // _CH, body,
                             jnp.zeros((_TILE, 1), jnp.float32))

    col = _fiota( (_TILE, _KP), 1)
    onehot = jnp.where((col == rank) & (rank < float(_K)), 1.0, 0.0)
    contrib = jax.lax.dot_general(
        onehot, aug_ref[...], (((0,), (0,)), ((), ())),
        preferred_element_type=jnp.float32)

    @pl.when(pid == 0)
    def _():
        out_ref[...] = jnp.zeros_like(out_ref)

    out_ref[...] += contrib

    @pl.when(pid == nprog - 1)
    def _():
        v = out_ref[...]
        cidx = jax.lax.broadcasted_iota(jnp.int32, (_KP, 8), 1)
        out_ref[...] = jnp.where(cidx < 6,
                                 jnp.clip(v, 0.0, _SCENE), v)


def _iou_kernel(aug_ref, augt_ref, out_ref):
    aug = aug_ref[...]
    augt = augt_ref[...]
    inter = jnp.ones((_BLK, _BLK), jnp.float32)
    vol_c = jnp.ones((_BLK, 1), jnp.float32)
    vol_r = jnp.ones((1, _BLK), jnp.float32)
    for d in range(3):
        lo_c, hi_c = aug[:, d:d + 1], aug[:, d + 3:d + 4]
        lo_r, hi_r = augt[d:d + 1, :], augt[d + 3:d + 4, :]
        ilo = jnp.maximum(lo_c, lo_r)
        ihi = jnp.minimum(hi_c, hi_r)
        inter = inter * jnp.clip(ihi - ilo, 0.0, None)
        vol_c = vol_c * (hi_c - lo_c)
        vol_r = vol_r * (hi_r - lo_r)
    union = vol_c + vol_r - inter
    out_ref[...] = inter / jnp.maximum(union, 1e-6)


def _nms_order_kernel(iou_ref, aug_ref, augt_ref, out_ref):
    lane_b = _fiota( (1, _BLK), 1)
    lane_k = _fiota( (1, _KP), 1)

    score_row = augt_ref[6:7, :]
    keep = jnp.where(score_row > 0.0, 1.0, 0.0)

    for b in range(_NBLK):
        base = b * _BLK
        kb0 = keep[:, base:base + _BLK]
        tile = iou_ref[base:base + _BLK, base:base + _BLK]

        def inner(i, kb):
            i_f = i.astype(jnp.float32)
            moh = jnp.where(lane_b == i_f, kb, 0.0)
            row = jax.lax.dot_general(
                moh, tile, (((1,), (0,)), ((), ())),
                preferred_element_type=jnp.float32)
            sup = (row > _THR) & (lane_b > i_f)
            return jnp.where(sup, 0.0, kb)

        kb = jax.lax.fori_loop(0, _BLK, inner, kb0)

        pieces = []
        if base > 0:
            pieces.append(keep[:, :base])
        pieces.append(kb)
        if base + _BLK < _KP:
            pieces.append(keep[:, base + _BLK:])
        keep = jnp.concatenate(pieces, axis=1)

        if base + _BLK < _KP:
            panel = jnp.where(iou_ref[base:base + _BLK, :] > _THR, 1.0, 0.0)
            contrib = jax.lax.dot_general(
                kb, panel, (((1,), (0,)), ((), ())),
                preferred_element_type=jnp.float32)
            supp = (contrib > 0.5) & (lane_k >= float(base + _BLK))
            keep = jnp.where(supp, 0.0, keep)

    real = jnp.where(lane_k < float(_K), 1.0, 0.0)
    notk = real * (1.0 - keep)
    nk = jnp.sum(keep, axis=1, keepdims=True)

    csk_parts, csnk_parts = [], []
    r_kp = _fiota( (_KP, _BLK), 0)
    c_blk = _fiota( (_KP, _BLK), 1)
    for b in range(_NBLK):
        ut = jnp.where(r_kp <= c_blk + float(b * _BLK), 1.0, 0.0)
        csk_parts.append(jax.lax.dot_general(
            keep, ut, (((1,), (0,)), ((), ())),
            preferred_element_type=jnp.float32))
        csnk_parts.append(jax.lax.dot_general(
            notk, ut, (((1,), (0,)), ((), ())),
            preferred_element_type=jnp.float32))
    csk = jnp.concatenate(csk_parts, axis=1)
    csnk = jnp.concatenate(csnk_parts, axis=1)
    pos = jnp.where(keep > 0.5, csk - 1.0, nk + csnk - 1.0)
    pos = jnp.where(lane_k < float(_K), pos, 4.0 * _KP)

    c_kp = _fiota( (_BLK, _KP), 1)
    r_blk = _fiota( (_BLK, _KP), 0)
    acc = jnp.zeros((_KP, 8), jnp.float32)
    for b in range(_NBLK):
        base = b * _BLK
        sel = (c_kp == r_blk + float(base))
        pos_col = jnp.sum(jnp.where(sel, pos, 0.0),
                          axis=1, keepdims=True)
        keep_col = jnp.sum(jnp.where(sel, keep, 0.0),
                           axis=1, keepdims=True)
        perm = jnp.where(c_kp == pos_col, 1.0, 0.0)
        aug_blk = aug_ref[base:base + _BLK, :]
        cidx = jax.lax.broadcasted_iota(jnp.int32, (_BLK, 8), 1)
        aug_blk = jnp.where(cidx == 7, keep_col, aug_blk)
        acc = acc + jax.lax.dot_general(
            perm, aug_blk, (((0,), (0,)), ((), ())),
            preferred_element_type=jnp.float32)
    out_ref[...] = acc


def kernel(boxes, scores):
    pad = _NP - _N_RAW
    boxes_p = jnp.pad(boxes, ((0, pad), (0, 0)))
    scores_p = jnp.pad(scores, (0, pad), constant_values=-1.0)
    aug_all = jnp.concatenate(
        [boxes_p, scores_p[:, None], jnp.zeros((_NP, 1), jnp.float32)],
        axis=1)

    cand = pl.pallas_call(
        _rank_select_kernel,
        grid=(_NP // _TILE,),
        in_specs=[
            pl.BlockSpec((_TILE, 1), lambda i: (i, 0)),
            pl.BlockSpec((1, _NP), lambda i: (0, 0)),
            pl.BlockSpec((_TILE, 8), lambda i: (i, 0)),
        ],
        out_specs=pl.BlockSpec((_KP, 8), lambda i: (0, 0)),
        out_shape=jax.ShapeDtypeStruct((_KP, 8), jnp.float32),
    )(scores_p[:, None], scores_p[None, :], aug_all)

    cand_t = cand.T

    iou = pl.pallas_call(
        _iou_kernel,
        grid=(_NBLK, _NBLK),
        in_specs=[
            pl.BlockSpec((_BLK, 8), lambda i, j: (i, 0)),
            pl.BlockSpec((8, _BLK), lambda i, j: (0, j)),
        ],
        out_specs=pl.BlockSpec((_BLK, _BLK), lambda i, j: (i, j)),
        out_shape=jax.ShapeDtypeStruct((_KP, _KP), jnp.float32),
    )(cand, cand_t)

    out = pl.pallas_call(
        _nms_order_kernel,
        in_specs=[
            pl.BlockSpec((_KP, _KP), lambda: (0, 0)),
            pl.BlockSpec((_KP, 8), lambda: (0, 0)),
            pl.BlockSpec((8, _KP), lambda: (0, 0)),
        ],
        out_specs=pl.BlockSpec((_KP, 8), lambda: (0, 0)),
        out_shape=jax.ShapeDtypeStruct((_KP, 8), jnp.float32),
    )(iou, cand, cand_t)

    proposals = out[:_K, :6]
    final_scores = jnp.where(out[:_K, 7] > 0.5, out[:_K, 6], -jnp.inf)
    return proposals, final_scores


# speedup vs baseline: 1.7373x; 1.4397x over previous
import jax
import jax.numpy as jnp
from jax.experimental import pallas as pl

_N_RAW = 20000
_NP = 20480
_TILE = 256
_CH = 512
_K = 1000
_KP = 1024
_BLK = 128
_NBLK = _KP // _BLK
_THR = 0.7
_SCENE = 128.0


def _fiota(shape, dim):
    return jax.lax.broadcasted_iota(jnp.int32, shape, dim).astype(jnp.float32)


def _rank_select_kernel(s_col_ref, s_row_ref, aug_ref, out_ref):
    pid = pl.program_id(0)
    nprog = pl.num_programs(0)
    s_col = s_col_ref[...]

    cd = pid // (_CH // _TILE)

    def ge_body(c, acc):
        s_chunk = s_row_ref[:, pl.ds(c * _CH, _CH)]
        return acc + jnp.where(s_chunk >= s_col, 1.0, 0.0)

    def gt_body(c, acc):
        s_chunk = s_row_ref[:, pl.ds(c * _CH, _CH)]
        return acc + jnp.where(s_chunk > s_col, 1.0, 0.0)

    acc = jnp.zeros((_TILE, _CH), jnp.float32)
    acc = jax.lax.fori_loop(0, cd, ge_body, acc)
    acc = jax.lax.fori_loop(cd + 1, _NP // _CH, gt_body, acc)

    off = cd * _CH
    s_chunk = s_row_ref[:, pl.ds(off, _CH)]
    j_glob = off.astype(jnp.float32) + _fiota((1, _CH), 1)
    i_glob = ((pid * _TILE).astype(jnp.float32)
              + _fiota((_TILE, 1), 0))
    gt = s_chunk > s_col
    tie = (s_chunk == s_col) & (j_glob < i_glob)
    acc = acc + jnp.where(gt | tie, 1.0, 0.0)
    rank = jnp.sum(acc, axis=1, keepdims=True)

    col = _fiota( (_TILE, _KP), 1)
    onehot = jnp.where((col == rank) & (rank < float(_K)), 1.0, 0.0)
    contrib = jax.lax.dot_general(
        onehot, aug_ref[...], (((0,), (0,)), ((), ())),
        preferred_element_type=jnp.float32)

    @pl.when(pid == 0)
    def _():
        out_ref[...] = jnp.zeros_like(out_ref)

    out_ref[...] += contrib

    @pl.when(pid == nprog - 1)
    def _():
        v = out_ref[...]
        cidx = jax.lax.broadcasted_iota(jnp.int32, (_KP, 8), 1)
        out_ref[...] = jnp.where(cidx < 6,
                                 jnp.clip(v, 0.0, _SCENE), v)


def _iou_kernel(aug_ref, augt_ref, out_ref):
    aug = aug_ref[...]
    augt = augt_ref[...]
    inter = jnp.ones((_BLK, _BLK), jnp.float32)
    vol_c = jnp.ones((_BLK, 1), jnp.float32)
    vol_r = jnp.ones((1, _BLK), jnp.float32)
    for d in range(3):
        lo_c, hi_c = aug[:, d:d + 1], aug[:, d + 3:d + 4]
        lo_r, hi_r = augt[d:d + 1, :], augt[d + 3:d + 4, :]
        ilo = jnp.maximum(lo_c, lo_r)
        ihi = jnp.minimum(hi_c, hi_r)
        inter = inter * jnp.clip(ihi - ilo, 0.0, None)
        vol_c = vol_c * (hi_c - lo_c)
        vol_r = vol_r * (hi_r - lo_r)
    union = vol_c + vol_r - inter
    out_ref[...] = inter / jnp.maximum(union, 1e-6)


def _nms_order_kernel(iou_ref, aug_ref, augt_ref, out_ref):
    lane_b = _fiota( (1, _BLK), 1)
    lane_k = _fiota( (1, _KP), 1)

    score_row = augt_ref[6:7, :]
    keep = jnp.where(score_row > 0.0, 1.0, 0.0)

    for b in range(_NBLK):
        base = b * _BLK
        kb0 = keep[:, base:base + _BLK]
        tile = iou_ref[base:base + _BLK, base:base + _BLK]

        def inner(i, kb):
            i_f = i.astype(jnp.float32)
            moh = jnp.where(lane_b == i_f, kb, 0.0)
            row = jax.lax.dot_general(
                moh, tile, (((1,), (0,)), ((), ())),
                preferred_element_type=jnp.float32)
            sup = (row > _THR) & (lane_b > i_f)
            return jnp.where(sup, 0.0, kb)

        kb = jax.lax.fori_loop(0, _BLK, inner, kb0)

        pieces = []
        if base > 0:
            pieces.append(keep[:, :base])
        pieces.append(kb)
        if base + _BLK < _KP:
            pieces.append(keep[:, base + _BLK:])
        keep = jnp.concatenate(pieces, axis=1)

        if base + _BLK < _KP:
            panel = jnp.where(iou_ref[base:base + _BLK, :] > _THR, 1.0, 0.0)
            contrib = jax.lax.dot_general(
                kb, panel, (((1,), (0,)), ((), ())),
                preferred_element_type=jnp.float32)
            supp = (contrib > 0.5) & (lane_k >= float(base + _BLK))
            keep = jnp.where(supp, 0.0, keep)

    real = jnp.where(lane_k < float(_K), 1.0, 0.0)
    notk = real * (1.0 - keep)
    nk = jnp.sum(keep, axis=1, keepdims=True)

    csk_parts, csnk_parts = [], []
    r_kp = _fiota( (_KP, _BLK), 0)
    c_blk = _fiota( (_KP, _BLK), 1)
    for b in range(_NBLK):
        ut = jnp.where(r_kp <= c_blk + float(b * _BLK), 1.0, 0.0)
        csk_parts.append(jax.lax.dot_general(
            keep, ut, (((1,), (0,)), ((), ())),
            preferred_element_type=jnp.float32))
        csnk_parts.append(jax.lax.dot_general(
            notk, ut, (((1,), (0,)), ((), ())),
            preferred_element_type=jnp.float32))
    csk = jnp.concatenate(csk_parts, axis=1)
    csnk = jnp.concatenate(csnk_parts, axis=1)
    pos = jnp.where(keep > 0.5, csk - 1.0, nk + csnk - 1.0)
    pos = jnp.where(lane_k < float(_K), pos, 4.0 * _KP)

    c_kp = _fiota( (_BLK, _KP), 1)
    r_blk = _fiota( (_BLK, _KP), 0)
    acc = jnp.zeros((_KP, 8), jnp.float32)
    for b in range(_NBLK):
        base = b * _BLK
        sel = (c_kp == r_blk + float(base))
        pos_col = jnp.sum(jnp.where(sel, pos, 0.0),
                          axis=1, keepdims=True)
        keep_col = jnp.sum(jnp.where(sel, keep, 0.0),
                           axis=1, keepdims=True)
        perm = jnp.where(c_kp == pos_col, 1.0, 0.0)
        aug_blk = aug_ref[base:base + _BLK, :]
        cidx = jax.lax.broadcasted_iota(jnp.int32, (_BLK, 8), 1)
        aug_blk = jnp.where(cidx == 7, keep_col, aug_blk)
        acc = acc + jax.lax.dot_general(
            perm, aug_blk, (((0,), (0,)), ((), ())),
            preferred_element_type=jnp.float32)
    out_ref[...] = acc


def kernel(boxes, scores):
    pad = _NP - _N_RAW
    boxes_p = jnp.pad(boxes, ((0, pad), (0, 0)))
    scores_p = jnp.pad(scores, (0, pad), constant_values=-1.0)
    aug_all = jnp.concatenate(
        [boxes_p, scores_p[:, None], jnp.zeros((_NP, 1), jnp.float32)],
        axis=1)

    cand = pl.pallas_call(
        _rank_select_kernel,
        grid=(_NP // _TILE,),
        in_specs=[
            pl.BlockSpec((_TILE, 1), lambda i: (i, 0)),
            pl.BlockSpec((1, _NP), lambda i: (0, 0)),
            pl.BlockSpec((_TILE, 8), lambda i: (i, 0)),
        ],
        out_specs=pl.BlockSpec((_KP, 8), lambda i: (0, 0)),
        out_shape=jax.ShapeDtypeStruct((_KP, 8), jnp.float32),
    )(scores_p[:, None], scores_p[None, :], aug_all)

    cand_t = cand.T

    iou = pl.pallas_call(
        _iou_kernel,
        grid=(_NBLK, _NBLK),
        in_specs=[
            pl.BlockSpec((_BLK, 8), lambda i, j: (i, 0)),
            pl.BlockSpec((8, _BLK), lambda i, j: (0, j)),
        ],
        out_specs=pl.BlockSpec((_BLK, _BLK), lambda i, j: (i, j)),
        out_shape=jax.ShapeDtypeStruct((_KP, _KP), jnp.float32),
    )(cand, cand_t)

    out = pl.pallas_call(
        _nms_order_kernel,
        in_specs=[
            pl.BlockSpec((_KP, _KP), lambda: (0, 0)),
            pl.BlockSpec((_KP, 8), lambda: (0, 0)),
            pl.BlockSpec((8, _KP), lambda: (0, 0)),
        ],
        out_specs=pl.BlockSpec((_KP, 8), lambda: (0, 0)),
        out_shape=jax.ShapeDtypeStruct((_KP, 8), jnp.float32),
    )(iou, cand, cand_t)

    proposals = out[:_K, :6]
    final_scores = jnp.where(out[:_K, 7] > 0.5, out[:_K, 6], -jnp.inf)
    return proposals, final_scores


# speedup vs baseline: 4.5401x; 2.6133x over previous
import jax
import jax.numpy as jnp
from jax.experimental import pallas as pl
from jax.experimental.pallas import tpu as pltpu

_N_RAW = 20000
_NP = 20480
_TILE = 256
_K = 1000
_KP = 1024
_BLK = 128
_NBLK = _KP // _BLK
_THR = 0.7
_SCENE = 128.0
_HI = jax.lax.Precision.HIGHEST


def _fiota(shape, dim):
    return jax.lax.broadcasted_iota(jnp.int32, shape, dim).astype(jnp.float32)


def _iiota(shape, dim):
    return jax.lax.broadcasted_iota(jnp.int32, shape, dim)


def _select_kernel(s2d_ref, s_col_ref, aug_ref, out_ref, sm_ref):
    pid = pl.program_id(0)

    @pl.when(pid == 0)
    def _():
        out_ref[...] = jnp.zeros_like(out_ref)
        sbits = jax.lax.bitcast_convert_type(s2d_ref[...], jnp.int32)
        x = jnp.int32(0)
        for b in range(30, -1, -1):
            cand = x | jnp.int32(1 << b)
            cnt = jnp.sum(jnp.where(sbits >= cand, 1.0, 0.0))
            x = jnp.where(cnt >= float(_K), cand, x)
        n_gt = jnp.sum(jnp.where(sbits > x, 1.0, 0.0))
        sm_ref[0, 0] = x
        sm_ref[0, 1] = (float(_K) - n_gt).astype(jnp.int32)
        sm_ref[0, 2] = 0
        sm_ref[0, 3] = 0

    v_bits = sm_ref[0, 0]
    need_ties = sm_ref[0, 1].astype(jnp.float32)
    base = sm_ref[0, 2].astype(jnp.float32)
    tie_base = sm_ref[0, 3].astype(jnp.float32)

    sb = jax.lax.bitcast_convert_type(s_col_ref[...], jnp.int32)
    is_gt = sb > v_bits
    is_tie = sb == v_bits
    gt_f = jnp.where(is_gt, 1.0, 0.0)
    tie_f = jnp.where(is_tie, 1.0, 0.0)

    r_t = _iiota((_TILE, _TILE), 0)
    c_t = _iiota((_TILE, _TILE), 1)
    tri = jnp.where(c_t <= r_t, 1.0, 0.0)

    tie_incl = jax.lax.dot_general(
        tri, tie_f, (((1,), (0,)), ((), ())),
        preferred_element_type=jnp.float32)
    tie_excl = tie_base + tie_incl - tie_f
    sel = jnp.where(is_gt | (is_tie & (tie_excl < need_ties)), 1.0, 0.0)

    sel_incl = jax.lax.dot_general(
        tri, sel, (((1,), (0,)), ((), ())),
        preferred_element_type=jnp.float32)
    slot = base + sel_incl - 1.0

    col = _fiota((_TILE, _KP), 1)
    onehot = jnp.where((col == slot) & (sel > 0.5), 1.0, 0.0)
    out_ref[...] += jax.lax.dot_general(
        onehot, aug_ref[...], (((0,), (0,)), ((), ())),
        preferred_element_type=jnp.float32, precision=_HI)

    sm_ref[0, 2] = (base + jnp.sum(sel)).astype(jnp.int32)
    sm_ref[0, 3] = (tie_base + jnp.sum(tie_f)).astype(jnp.int32)


def _sort_nms_kernel(aug_ref, augt_ref, out_ref, iou_ref):
    aug = aug_ref[...]
    augt = augt_ref[...]
    lane_b = _fiota((1, _BLK), 1)
    lane_k = _fiota((1, _KP), 1)

    real_r = lane_k < float(_K)
    s_row = jnp.where(real_r, augt[6:7, :], -1.0)
    i_row = jnp.where(real_r, augt[7:8, :], 30000.0 + lane_k)

    rank_parts = []
    for b in range(_NBLK):
        base = b * _BLK
        rc = _fiota((_BLK, 1), 0) + float(base)
        s_col = jnp.where(rc < float(_K), aug[base:base + _BLK, 6:7], -1.0)
        i_col = jnp.where(rc < float(_K), aug[base:base + _BLK, 7:8],
                          30000.0 + rc)
        gt = s_row > s_col
        tie = (s_row == s_col) & (i_row < i_col)
        rank_parts.append(jnp.sum(jnp.where(gt | tie, 1.0, 0.0),
                                  axis=1, keepdims=True))
    rank = jnp.concatenate(rank_parts, axis=0)

    c_kp = _fiota((_BLK, _KP), 1)
    cand = jnp.zeros((_KP, 8), jnp.float32)
    candt = jnp.zeros((8, _KP), jnp.float32)
    for b in range(_NBLK):
        base = b * _BLK
        perm = jnp.where(c_kp == rank[base:base + _BLK, :], 1.0, 0.0)
        cand = cand + jax.lax.dot_general(
            perm, aug[base:base + _BLK, :], (((0,), (0,)), ((), ())),
            preferred_element_type=jnp.float32, precision=_HI)
        candt = candt + jax.lax.dot_general(
            augt[:, base:base + _BLK], perm, (((1,), (0,)), ((), ())),
            preferred_element_type=jnp.float32, precision=_HI)

    cidx8 = _iiota((_KP, 8), 1)
    cand = jnp.where(cidx8 < 6, jnp.clip(cand, 0.0, _SCENE), cand)
    ridx8 = _iiota((8, _KP), 0)
    candt = jnp.where(ridx8 < 6, jnp.clip(candt, 0.0, _SCENE), candt)

    for b in range(_NBLK):
        base = b * _BLK
        blk = cand[base:base + _BLK, :]
        inter = jnp.ones((_BLK, _KP), jnp.float32)
        vol_c = jnp.ones((_BLK, 1), jnp.float32)
        vol_r = jnp.ones((1, _KP), jnp.float32)
        for d in range(3):
            lo_c, hi_c = blk[:, d:d + 1], blk[:, d + 3:d + 4]
            lo_r, hi_r = candt[d:d + 1, :], candt[d + 3:d + 4, :]
            inter = inter * jnp.clip(jnp.minimum(hi_c, hi_r)
                                     - jnp.maximum(lo_c, lo_r), 0.0, None)
            vol_c = vol_c * (hi_c - lo_c)
            vol_r = vol_r * (hi_r - lo_r)
        union = vol_c + vol_r - inter
        iou_ref[base:base + _BLK, :] = inter / jnp.maximum(union, 1e-6)

    score_row = candt[6:7, :]
    keep = jnp.where(score_row > 0.0, 1.0, 0.0)

    for b in range(_NBLK):
        base = b * _BLK
        kb0 = keep[:, base:base + _BLK]
        tile = jnp.where(
            iou_ref[base:base + _BLK, base:base + _BLK] > _THR, 1.0, 0.0)

        def inner(i, kb):
            i_f = i.astype(jnp.float32)
            moh = jnp.where(lane_b == i_f, kb, 0.0)
            row = jax.lax.dot_general(
                moh, tile, (((1,), (0,)), ((), ())),
                preferred_element_type=jnp.float32)
            sup = (row > 0.5) & (lane_b > i_f)
            return jnp.where(sup, 0.0, kb)

        kb = jax.lax.fori_loop(0, _BLK, inner, kb0)

        pieces = []
        if base > 0:
            pieces.append(keep[:, :base])
        pieces.append(kb)
        if base + _BLK < _KP:
            pieces.append(keep[:, base + _BLK:])
        keep = jnp.concatenate(pieces, axis=1)

        if base + _BLK < _KP:
            panel = jnp.where(iou_ref[base:base + _BLK, :] > _THR, 1.0, 0.0)
            contrib = jax.lax.dot_general(
                kb, panel, (((1,), (0,)), ((), ())),
                preferred_element_type=jnp.float32)
            supp = (contrib > 0.5) & (lane_k >= float(base + _BLK))
            keep = jnp.where(supp, 0.0, keep)

    real = jnp.where(lane_k < float(_K), 1.0, 0.0)
    notk = real * (1.0 - keep)
    nk = jnp.sum(keep, axis=1, keepdims=True)

    csk_parts, csnk_parts = [], []
    r_kp = _fiota((_KP, _BLK), 0)
    c_blk = _fiota((_KP, _BLK), 1)
    for b in range(_NBLK):
        ut = jnp.where(r_kp <= c_blk + float(b * _BLK), 1.0, 0.0)
        csk_parts.append(jax.lax.dot_general(
            keep, ut, (((1,), (0,)), ((), ())),
            preferred_element_type=jnp.float32))
        csnk_parts.append(jax.lax.dot_general(
            notk, ut, (((1,), (0,)), ((), ())),
            preferred_element_type=jnp.float32))
    csk = jnp.concatenate(csk_parts, axis=1)
    csnk = jnp.concatenate(csnk_parts, axis=1)
    pos = jnp.where(keep > 0.5, csk - 1.0, nk + csnk - 1.0)
    pos = jnp.where(lane_k < float(_K), pos, 4.0 * _KP)

    r_blk = _fiota((_BLK, _KP), 0)
    acc = jnp.zeros((_KP, 8), jnp.float32)
    for b in range(_NBLK):
        base = b * _BLK
        sel = (c_kp == r_blk + float(base))
        pos_col = jnp.sum(jnp.where(sel, pos, 0.0),
                          axis=1, keepdims=True)
        keep_col = jnp.sum(jnp.where(sel, keep, 0.0),
                           axis=1, keepdims=True)
        perm = jnp.where(c_kp == pos_col, 1.0, 0.0)
        aug_blk = cand[base:base + _BLK, :]
        cidx = _iiota((_BLK, 8), 1)
        aug_blk = jnp.where(cidx == 7, keep_col, aug_blk)
        acc = acc + jax.lax.dot_general(
            perm, aug_blk, (((0,), (0,)), ((), ())),
            preferred_element_type=jnp.float32, precision=_HI)
    out_ref[...] = acc


def kernel(boxes, scores):
    pad = _NP - _N_RAW
    boxes_p = jnp.pad(boxes, ((0, pad), (0, 0)))
    scores_p = jnp.pad(scores, (0, pad), constant_values=-1.0)
    idx_col = jnp.arange(_NP, dtype=jnp.float32)[:, None]
    aug_all = jnp.concatenate(
        [boxes_p, scores_p[:, None], idx_col], axis=1)

    cand = pl.pallas_call(
        _select_kernel,
        grid=(_NP // _TILE,),
        in_specs=[
            pl.BlockSpec((_NP // 128, 128), lambda i: (0, 0)),
            pl.BlockSpec((_TILE, 1), lambda i: (i, 0)),
            pl.BlockSpec((_TILE, 8), lambda i: (i, 0)),
        ],
        out_specs=pl.BlockSpec((_KP, 8), lambda i: (0, 0)),
        out_shape=jax.ShapeDtypeStruct((_KP, 8), jnp.float32),
        scratch_shapes=[pltpu.SMEM((1, 4), jnp.int32)],
    )(scores_p.reshape(_NP // 128, 128), scores_p[:, None], aug_all)

    out = pl.pallas_call(
        _sort_nms_kernel,
        in_specs=[
            pl.BlockSpec((_KP, 8), lambda: (0, 0)),
            pl.BlockSpec((8, _KP), lambda: (0, 0)),
        ],
        out_specs=pl.BlockSpec((_KP, 8), lambda: (0, 0)),
        out_shape=jax.ShapeDtypeStruct((_KP, 8), jnp.float32),
        scratch_shapes=[pltpu.VMEM((_KP, _KP), jnp.float32)],
    )(cand, cand.T)

    proposals = out[:_K, :6]
    final_scores = jnp.where(out[:_K, 7] > 0.5, out[:_K, 6], -jnp.inf)
    return proposals, final_scores


# speedup vs baseline: 4.5582x; 1.0040x over previous
import jax
import jax.numpy as jnp
from jax.experimental import pallas as pl
from jax.experimental.pallas import tpu as pltpu

_N_RAW = 20000
_NP = 20480
_TILE = 256
_K = 1000
_KP = 1024
_BLK = 128
_NBLK = _KP // _BLK
_THR = 0.7
_SCENE = 128.0
_HI = jax.lax.Precision.HIGHEST


def _fiota(shape, dim):
    return jax.lax.broadcasted_iota(jnp.int32, shape, dim).astype(jnp.float32)


def _iiota(shape, dim):
    return jax.lax.broadcasted_iota(jnp.int32, shape, dim)


def _select_kernel(s2d_ref, s_col_ref, aug_ref, out_ref, sm_ref):
    pid = pl.program_id(0)

    @pl.when(pid == 0)
    def _():
        out_ref[...] = jnp.zeros_like(out_ref)
        sbits = jax.lax.bitcast_convert_type(s2d_ref[...], jnp.int32)
        x = jnp.int32(0)
        for b in range(30, -1, -1):
            cand = x | jnp.int32(1 << b)
            cnt = jnp.sum(jnp.where(sbits >= cand, 1.0, 0.0))
            x = jnp.where(cnt >= float(_K), cand, x)
        n_gt = jnp.sum(jnp.where(sbits > x, 1.0, 0.0))
        sm_ref[0, 0] = x
        sm_ref[0, 1] = (float(_K) - n_gt).astype(jnp.int32)
        sm_ref[0, 2] = 0
        sm_ref[0, 3] = 0

    v_bits = sm_ref[0, 0]
    need_ties = sm_ref[0, 1].astype(jnp.float32)
    base = sm_ref[0, 2].astype(jnp.float32)
    tie_base = sm_ref[0, 3].astype(jnp.float32)

    sb = jax.lax.bitcast_convert_type(s_col_ref[...], jnp.int32)
    is_gt = sb > v_bits
    is_tie = sb == v_bits
    gt_f = jnp.where(is_gt, 1.0, 0.0)
    tie_f = jnp.where(is_tie, 1.0, 0.0)

    r_t = _iiota((_TILE, _TILE), 0)
    c_t = _iiota((_TILE, _TILE), 1)
    tri = jnp.where(c_t <= r_t, 1.0, 0.0)

    tie_incl = jax.lax.dot_general(
        tri, tie_f, (((1,), (0,)), ((), ())),
        preferred_element_type=jnp.float32)
    tie_excl = tie_base + tie_incl - tie_f
    sel = jnp.where(is_gt | (is_tie & (tie_excl < need_ties)), 1.0, 0.0)

    sel_incl = jax.lax.dot_general(
        tri, sel, (((1,), (0,)), ((), ())),
        preferred_element_type=jnp.float32)
    slot = base + sel_incl - 1.0
    slot_i = jnp.where(sel > 0.5, slot, -1.0).astype(jnp.int32)

    col = _iiota((_TILE, _KP), 1)
    onehot = jnp.where(col == slot_i, 1.0, 0.0)
    out_ref[...] += jax.lax.dot_general(
        onehot, aug_ref[...], (((0,), (0,)), ((), ())),
        preferred_element_type=jnp.float32, precision=_HI)

    sm_ref[0, 2] = (base + jnp.sum(sel)).astype(jnp.int32)
    sm_ref[0, 3] = (tie_base + jnp.sum(tie_f)).astype(jnp.int32)


def _sort_nms_kernel(aug_ref, augt_ref, out_ref, iou_ref):
    aug = aug_ref[...]
    augt = augt_ref[...]
    lane_b = _fiota((1, _BLK), 1)
    lane_k = _fiota((1, _KP), 1)

    real_r = lane_k < float(_K)
    s_row = jnp.where(real_r, augt[6:7, :], -1.0)
    i_row = jnp.where(real_r, augt[7:8, :], 30000.0 + lane_k)

    rank_parts = []
    for b in range(_NBLK):
        base = b * _BLK
        rc = _fiota((_BLK, 1), 0) + float(base)
        s_col = jnp.where(rc < float(_K), aug[base:base + _BLK, 6:7], -1.0)
        i_col = jnp.where(rc < float(_K), aug[base:base + _BLK, 7:8],
                          30000.0 + rc)
        gt = s_row > s_col
        tie = (s_row == s_col) & (i_row < i_col)
        rank_parts.append(jnp.sum(jnp.where(gt | tie, 1.0, 0.0),
                                  axis=1, keepdims=True))
    rank = jnp.concatenate(rank_parts, axis=0)

    c_kp = _fiota((_BLK, _KP), 1)
    cand = jnp.zeros((_KP, 8), jnp.float32)
    candt = jnp.zeros((8, _KP), jnp.float32)
    for b in range(_NBLK):
        base = b * _BLK
        perm = jnp.where(c_kp == rank[base:base + _BLK, :], 1.0, 0.0)
        cand = cand + jax.lax.dot_general(
            perm, aug[base:base + _BLK, :], (((0,), (0,)), ((), ())),
            preferred_element_type=jnp.float32, precision=_HI)
        candt = candt + jax.lax.dot_general(
            augt[:, base:base + _BLK], perm, (((1,), (0,)), ((), ())),
            preferred_element_type=jnp.float32, precision=_HI)

    cidx8 = _iiota((_KP, 8), 1)
    cand = jnp.where(cidx8 < 6, jnp.clip(cand, 0.0, _SCENE), cand)
    ridx8 = _iiota((8, _KP), 0)
    candt = jnp.where(ridx8 < 6, jnp.clip(candt, 0.0, _SCENE), candt)

    for b in range(_NBLK):
        base = b * _BLK
        blk = cand[base:base + _BLK, :]
        inter = jnp.ones((_BLK, _KP), jnp.float32)
        vol_c = jnp.ones((_BLK, 1), jnp.float32)
        vol_r = jnp.ones((1, _KP), jnp.float32)
        for d in range(3):
            lo_c, hi_c = blk[:, d:d + 1], blk[:, d + 3:d + 4]
            lo_r, hi_r = candt[d:d + 1, :], candt[d + 3:d + 4, :]
            inter = inter * jnp.clip(jnp.minimum(hi_c, hi_r)
                                     - jnp.maximum(lo_c, lo_r), 0.0, None)
            vol_c = vol_c * (hi_c - lo_c)
            vol_r = vol_r * (hi_r - lo_r)
        union = vol_c + vol_r - inter
        iou_ref[base:base + _BLK, :] = inter / jnp.maximum(union, 1e-6)

    score_row = candt[6:7, :]
    keep = jnp.where(score_row > 0.0, 1.0, 0.0)

    for b in range(_NBLK):
        base = b * _BLK
        kb0 = keep[:, base:base + _BLK]
        tile = jnp.where(
            iou_ref[base:base + _BLK, base:base + _BLK] > _THR, 1.0, 0.0)

        def inner(i, kb):
            i_f = i.astype(jnp.float32)
            moh = jnp.where(lane_b == i_f, kb, 0.0)
            row = jax.lax.dot_general(
                moh, tile, (((1,), (0,)), ((), ())),
                preferred_element_type=jnp.float32)
            sup = (row > 0.5) & (lane_b > i_f)
            return jnp.where(sup, 0.0, kb)

        kb = jax.lax.fori_loop(0, _BLK, inner, kb0)

        pieces = []
        if base > 0:
            pieces.append(keep[:, :base])
        pieces.append(kb)
        if base + _BLK < _KP:
            pieces.append(keep[:, base + _BLK:])
        keep = jnp.concatenate(pieces, axis=1)

        if base + _BLK < _KP:
            panel = jnp.where(iou_ref[base:base + _BLK, :] > _THR, 1.0, 0.0)
            contrib = jax.lax.dot_general(
                kb, panel, (((1,), (0,)), ((), ())),
                preferred_element_type=jnp.float32)
            supp = (contrib > 0.5) & (lane_k >= float(base + _BLK))
            keep = jnp.where(supp, 0.0, keep)

    real = jnp.where(lane_k < float(_K), 1.0, 0.0)
    notk = real * (1.0 - keep)
    nk = jnp.sum(keep, axis=1, keepdims=True)

    csk_parts, csnk_parts = [], []
    r_kp = _fiota((_KP, _BLK), 0)
    c_blk = _fiota((_KP, _BLK), 1)
    for b in range(_NBLK):
        ut = jnp.where(r_kp <= c_blk + float(b * _BLK), 1.0, 0.0)
        csk_parts.append(jax.lax.dot_general(
            keep, ut, (((1,), (0,)), ((), ())),
            preferred_element_type=jnp.float32))
        csnk_parts.append(jax.lax.dot_general(
            notk, ut, (((1,), (0,)), ((), ())),
            preferred_element_type=jnp.float32))
    csk = jnp.concatenate(csk_parts, axis=1)
    csnk = jnp.concatenate(csnk_parts, axis=1)
    pos = jnp.where(keep > 0.5, csk - 1.0, nk + csnk - 1.0)
    pos = jnp.where(lane_k < float(_K), pos, 4.0 * _KP)

    r_blk = _fiota((_BLK, _KP), 0)
    acc = jnp.zeros((_KP, 8), jnp.float32)
    for b in range(_NBLK):
        base = b * _BLK
        sel = (c_kp == r_blk + float(base))
        pos_col = jnp.sum(jnp.where(sel, pos, 0.0),
                          axis=1, keepdims=True)
        keep_col = jnp.sum(jnp.where(sel, keep, 0.0),
                           axis=1, keepdims=True)
        perm = jnp.where(c_kp == pos_col, 1.0, 0.0)
        aug_blk = cand[base:base + _BLK, :]
        cidx = _iiota((_BLK, 8), 1)
        aug_blk = jnp.where(cidx == 7, keep_col, aug_blk)
        acc = acc + jax.lax.dot_general(
            perm, aug_blk, (((0,), (0,)), ((), ())),
            preferred_element_type=jnp.float32, precision=_HI)
    out_ref[...] = acc


def kernel(boxes, scores):
    pad = _NP - _N_RAW
    boxes_p = jnp.pad(boxes, ((0, pad), (0, 0)))
    scores_p = jnp.pad(scores, (0, pad), constant_values=-1.0)
    idx_col = jnp.arange(_NP, dtype=jnp.float32)[:, None]
    aug_all = jnp.concatenate(
        [boxes_p, scores_p[:, None], idx_col], axis=1)

    cand = pl.pallas_call(
        _select_kernel,
        grid=(_NP // _TILE,),
        in_specs=[
            pl.BlockSpec((_NP // 128, 128), lambda i: (0, 0)),
            pl.BlockSpec((_TILE, 1), lambda i: (i, 0)),
            pl.BlockSpec((_TILE, 8), lambda i: (i, 0)),
        ],
        out_specs=pl.BlockSpec((_KP, 8), lambda i: (0, 0)),
        out_shape=jax.ShapeDtypeStruct((_KP, 8), jnp.float32),
        scratch_shapes=[pltpu.SMEM((1, 4), jnp.int32)],
    )(scores_p.reshape(_NP // 128, 128), scores_p[:, None], aug_all)

    out = pl.pallas_call(
        _sort_nms_kernel,
        in_specs=[
            pl.BlockSpec((_KP, 8), lambda: (0, 0)),
            pl.BlockSpec((8, _KP), lambda: (0, 0)),
        ],
        out_specs=pl.BlockSpec((_KP, 8), lambda: (0, 0)),
        out_shape=jax.ShapeDtypeStruct((_KP, 8), jnp.float32),
        scratch_shapes=[pltpu.VMEM((_KP, _KP), jnp.float32)],
    )(cand, cand.T)

    proposals = out[:_K, :6]
    final_scores = jnp.where(out[:_K, 7] > 0.5, out[:_K, 6], -jnp.inf)
    return proposals, final_scores


# speedup vs baseline: 4.8558x; 1.0653x over previous
import jax
import jax.numpy as jnp
from jax.experimental import pallas as pl
from jax.experimental.pallas import tpu as pltpu

_N_RAW = 20000
_NP = 20480
_TILE = 512
_K = 1000
_KP = 1024
_BLK = 128
_NBLK = _KP // _BLK
_THR = 0.7
_SCENE = 128.0
_HI = jax.lax.Precision.HIGHEST


def _fiota(shape, dim):
    return jax.lax.broadcasted_iota(jnp.int32, shape, dim).astype(jnp.float32)


def _iiota(shape, dim):
    return jax.lax.broadcasted_iota(jnp.int32, shape, dim)


def _select_kernel(s2d_ref, s_col_ref, aug_ref, out_ref, sm_ref):
    pid = pl.program_id(0)

    @pl.when(pid == 0)
    def _():
        out_ref[...] = jnp.zeros_like(out_ref)
        sbits = jax.lax.bitcast_convert_type(s2d_ref[...], jnp.int32)
        x = jnp.int32(0)
        for b in range(30, -1, -1):
            cand = x | jnp.int32(1 << b)
            cnt = jnp.sum(jnp.where(sbits >= cand, 1.0, 0.0))
            x = jnp.where(cnt >= float(_K), cand, x)
        n_gt = jnp.sum(jnp.where(sbits > x, 1.0, 0.0))
        sm_ref[0, 0] = x
        sm_ref[0, 1] = (float(_K) - n_gt).astype(jnp.int32)
        sm_ref[0, 2] = 0
        sm_ref[0, 3] = 0

    v_bits = sm_ref[0, 0]
    need_ties = sm_ref[0, 1].astype(jnp.float32)
    base = sm_ref[0, 2].astype(jnp.float32)
    tie_base = sm_ref[0, 3].astype(jnp.float32)

    sb = jax.lax.bitcast_convert_type(s_col_ref[...], jnp.int32)
    is_gt = sb > v_bits
    is_tie = sb == v_bits
    gt_f = jnp.where(is_gt, 1.0, 0.0)
    tie_f = jnp.where(is_tie, 1.0, 0.0)

    r_t = _iiota((_TILE, _TILE), 0)
    c_t = _iiota((_TILE, _TILE), 1)
    tri = jnp.where(c_t <= r_t, 1.0, 0.0)

    tie_incl = jax.lax.dot_general(
        tri, tie_f, (((1,), (0,)), ((), ())),
        preferred_element_type=jnp.float32)
    tie_excl = tie_base + tie_incl - tie_f
    sel = jnp.where(is_gt | (is_tie & (tie_excl < need_ties)), 1.0, 0.0)

    sel_incl = jax.lax.dot_general(
        tri, sel, (((1,), (0,)), ((), ())),
        preferred_element_type=jnp.float32)
    slot = base + sel_incl - 1.0
    slot_i = jnp.where(sel > 0.5, slot, -1.0).astype(jnp.int32)

    col = _iiota((_TILE, _KP), 1)
    onehot = jnp.where(col == slot_i, 1.0, 0.0)
    out_ref[...] += jax.lax.dot_general(
        onehot, aug_ref[...], (((0,), (0,)), ((), ())),
        preferred_element_type=jnp.float32, precision=_HI)

    sm_ref[0, 2] = (base + jnp.sum(sel)).astype(jnp.int32)
    sm_ref[0, 3] = (tie_base + jnp.sum(tie_f)).astype(jnp.int32)


def _sort_nms_kernel(aug_ref, augt_ref, out_ref, iou_ref):
    aug = aug_ref[...]
    augt = augt_ref[...]
    lane_b = _fiota((1, _BLK), 1)
    lane_k = _fiota((1, _KP), 1)

    real_r = lane_k < float(_K)
    s_row = jnp.where(real_r, augt[6:7, :], -1.0)
    i_row = jnp.where(real_r, augt[7:8, :], 30000.0 + lane_k)

    rank_parts = []
    for b in range(_NBLK):
        base = b * _BLK
        rc = _fiota((_BLK, 1), 0) + float(base)
        s_col = jnp.where(rc < float(_K), aug[base:base + _BLK, 6:7], -1.0)
        i_col = jnp.where(rc < float(_K), aug[base:base + _BLK, 7:8],
                          30000.0 + rc)
        gt = s_row > s_col
        tie = (s_row == s_col) & (i_row < i_col)
        rank_parts.append(jnp.sum(jnp.where(gt | tie, 1.0, 0.0),
                                  axis=1, keepdims=True))
    rank = jnp.concatenate(rank_parts, axis=0)

    c_kp = _fiota((_BLK, _KP), 1)
    cand = jnp.zeros((_KP, 8), jnp.float32)
    candt = jnp.zeros((8, _KP), jnp.float32)
    for b in range(_NBLK):
        base = b * _BLK
        perm = jnp.where(c_kp == rank[base:base + _BLK, :], 1.0, 0.0)
        cand = cand + jax.lax.dot_general(
            perm, aug[base:base + _BLK, :], (((0,), (0,)), ((), ())),
            preferred_element_type=jnp.float32, precision=_HI)
        candt = candt + jax.lax.dot_general(
            augt[:, base:base + _BLK], perm, (((1,), (0,)), ((), ())),
            preferred_element_type=jnp.float32, precision=_HI)

    cidx8 = _iiota((_KP, 8), 1)
    cand = jnp.where(cidx8 < 6, jnp.clip(cand, 0.0, _SCENE), cand)
    ridx8 = _iiota((8, _KP), 0)
    candt = jnp.where(ridx8 < 6, jnp.clip(candt, 0.0, _SCENE), candt)

    for b in range(_NBLK):
        base = b * _BLK
        blk = cand[base:base + _BLK, :]
        inter = jnp.ones((_BLK, _KP), jnp.float32)
        vol_c = jnp.ones((_BLK, 1), jnp.float32)
        vol_r = jnp.ones((1, _KP), jnp.float32)
        for d in range(3):
            lo_c, hi_c = blk[:, d:d + 1], blk[:, d + 3:d + 4]
            lo_r, hi_r = candt[d:d + 1, :], candt[d + 3:d + 4, :]
            inter = inter * jnp.clip(jnp.minimum(hi_c, hi_r)
                                     - jnp.maximum(lo_c, lo_r), 0.0, None)
            vol_c = vol_c * (hi_c - lo_c)
            vol_r = vol_r * (hi_r - lo_r)
        union = vol_c + vol_r - inter
        iou_ref[base:base + _BLK, :] = inter / jnp.maximum(union, 1e-6)

    score_row = candt[6:7, :]
    keep = jnp.where(score_row > 0.0, 1.0, 0.0)

    for b in range(_NBLK):
        base = b * _BLK
        kb0 = keep[:, base:base + _BLK]
        tile = jnp.where(
            iou_ref[base:base + _BLK, base:base + _BLK] > _THR, 1.0, 0.0)

        def inner(i, kb):
            i_f = i.astype(jnp.float32)
            moh = jnp.where(lane_b == i_f, kb, 0.0)
            row = jax.lax.dot_general(
                moh, tile, (((1,), (0,)), ((), ())),
                preferred_element_type=jnp.float32)
            sup = (row > 0.5) & (lane_b > i_f)
            return jnp.where(sup, 0.0, kb)

        kb = jax.lax.fori_loop(0, _BLK, inner, kb0)

        pieces = []
        if base > 0:
            pieces.append(keep[:, :base])
        pieces.append(kb)
        if base + _BLK < _KP:
            pieces.append(keep[:, base + _BLK:])
        keep = jnp.concatenate(pieces, axis=1)

        if base + _BLK < _KP:
            panel = jnp.where(iou_ref[base:base + _BLK, :] > _THR, 1.0, 0.0)
            contrib = jax.lax.dot_general(
                kb, panel, (((1,), (0,)), ((), ())),
                preferred_element_type=jnp.float32)
            supp = (contrib > 0.5) & (lane_k >= float(base + _BLK))
            keep = jnp.where(supp, 0.0, keep)

    real = jnp.where(lane_k < float(_K), 1.0, 0.0)
    notk = real * (1.0 - keep)
    nk = jnp.sum(keep, axis=1, keepdims=True)

    csk_parts, csnk_parts = [], []
    r_kp = _fiota((_KP, _BLK), 0)
    c_blk = _fiota((_KP, _BLK), 1)
    for b in range(_NBLK):
        ut = jnp.where(r_kp <= c_blk + float(b * _BLK), 1.0, 0.0)
        csk_parts.append(jax.lax.dot_general(
            keep, ut, (((1,), (0,)), ((), ())),
            preferred_element_type=jnp.float32))
        csnk_parts.append(jax.lax.dot_general(
            notk, ut, (((1,), (0,)), ((), ())),
            preferred_element_type=jnp.float32))
    csk = jnp.concatenate(csk_parts, axis=1)
    csnk = jnp.concatenate(csnk_parts, axis=1)
    pos = jnp.where(keep > 0.5, csk - 1.0, nk + csnk - 1.0)
    pos = jnp.where(lane_k < float(_K), pos, 4.0 * _KP)

    r_blk = _fiota((_BLK, _KP), 0)
    acc = jnp.zeros((_KP, 8), jnp.float32)
    for b in range(_NBLK):
        base = b * _BLK
        sel = (c_kp == r_blk + float(base))
        pos_col = jnp.sum(jnp.where(sel, pos, 0.0),
                          axis=1, keepdims=True)
        keep_col = jnp.sum(jnp.where(sel, keep, 0.0),
                           axis=1, keepdims=True)
        perm = jnp.where(c_kp == pos_col, 1.0, 0.0)
        aug_blk = cand[base:base + _BLK, :]
        cidx = _iiota((_BLK, 8), 1)
        aug_blk = jnp.where(cidx == 7, keep_col, aug_blk)
        acc = acc + jax.lax.dot_general(
            perm, aug_blk, (((0,), (0,)), ((), ())),
            preferred_element_type=jnp.float32, precision=_HI)
    out_ref[...] = acc


def kernel(boxes, scores):
    pad = _NP - _N_RAW
    boxes_p = jnp.pad(boxes, ((0, pad), (0, 0)))
    scores_p = jnp.pad(scores, (0, pad), constant_values=-1.0)
    idx_col = jnp.arange(_NP, dtype=jnp.float32)[:, None]
    aug_all = jnp.concatenate(
        [boxes_p, scores_p[:, None], idx_col], axis=1)

    cand = pl.pallas_call(
        _select_kernel,
        grid=(_NP // _TILE,),
        in_specs=[
            pl.BlockSpec((_NP // 128, 128), lambda i: (0, 0)),
            pl.BlockSpec((_TILE, 1), lambda i: (i, 0)),
            pl.BlockSpec((_TILE, 8), lambda i: (i, 0)),
        ],
        out_specs=pl.BlockSpec((_KP, 8), lambda i: (0, 0)),
        out_shape=jax.ShapeDtypeStruct((_KP, 8), jnp.float32),
        scratch_shapes=[pltpu.SMEM((1, 4), jnp.int32)],
    )(scores_p.reshape(_NP // 128, 128), scores_p[:, None], aug_all)

    out = pl.pallas_call(
        _sort_nms_kernel,
        in_specs=[
            pl.BlockSpec((_KP, 8), lambda: (0, 0)),
            pl.BlockSpec((8, _KP), lambda: (0, 0)),
        ],
        out_specs=pl.BlockSpec((_KP, 8), lambda: (0, 0)),
        out_shape=jax.ShapeDtypeStruct((_KP, 8), jnp.float32),
        scratch_shapes=[pltpu.VMEM((_KP, _KP), jnp.float32)],
    )(cand, cand.T)

    proposals = out[:_K, :6]
    final_scores = jnp.where(out[:_K, 7] > 0.5, out[:_K, 6], -jnp.inf)
    return proposals, final_scores


# speedup vs baseline: 5.0104x; 1.0318x over previous
import jax
import jax.numpy as jnp
from jax.experimental import pallas as pl
from jax.experimental.pallas import tpu as pltpu

_N_RAW = 20000
_NP = 20480
_TILE = 1024
_K = 1000
_KP = 1024
_BLK = 128
_NBLK = _KP // _BLK
_THR = 0.7
_SCENE = 128.0
_HI = jax.lax.Precision.HIGHEST


def _fiota(shape, dim):
    return jax.lax.broadcasted_iota(jnp.int32, shape, dim).astype(jnp.float32)


def _iiota(shape, dim):
    return jax.lax.broadcasted_iota(jnp.int32, shape, dim)


def _select_kernel(s2d_ref, s_col_ref, aug_ref, out_ref, sm_ref):
    pid = pl.program_id(0)

    @pl.when(pid == 0)
    def _():
        out_ref[...] = jnp.zeros_like(out_ref)
        sbits = jax.lax.bitcast_convert_type(s2d_ref[...], jnp.int32)
        x = jnp.int32(0)
        for b in range(30, -1, -1):
            cand = x | jnp.int32(1 << b)
            cnt = jnp.sum(jnp.where(sbits >= cand, 1.0, 0.0))
            x = jnp.where(cnt >= float(_K), cand, x)
        n_gt = jnp.sum(jnp.where(sbits > x, 1.0, 0.0))
        sm_ref[0, 0] = x
        sm_ref[0, 1] = (float(_K) - n_gt).astype(jnp.int32)
        sm_ref[0, 2] = 0
        sm_ref[0, 3] = 0

    v_bits = sm_ref[0, 0]
    need_ties = sm_ref[0, 1].astype(jnp.float32)
    base = sm_ref[0, 2].astype(jnp.float32)
    tie_base = sm_ref[0, 3].astype(jnp.float32)

    sb = jax.lax.bitcast_convert_type(s_col_ref[...], jnp.int32)
    is_gt = sb > v_bits
    is_tie = sb == v_bits
    gt_f = jnp.where(is_gt, 1.0, 0.0)
    tie_f = jnp.where(is_tie, 1.0, 0.0)

    r_t = _iiota((_TILE, _TILE), 0)
    c_t = _iiota((_TILE, _TILE), 1)
    tri = jnp.where(c_t <= r_t, 1.0, 0.0)

    tie_incl = jax.lax.dot_general(
        tri, tie_f, (((1,), (0,)), ((), ())),
        preferred_element_type=jnp.float32)
    tie_excl = tie_base + tie_incl - tie_f
    sel = jnp.where(is_gt | (is_tie & (tie_excl < need_ties)), 1.0, 0.0)

    sel_incl = jax.lax.dot_general(
        tri, sel, (((1,), (0,)), ((), ())),
        preferred_element_type=jnp.float32)
    slot = base + sel_incl - 1.0
    slot_i = jnp.where(sel > 0.5, slot, -1.0).astype(jnp.int32)

    col = _iiota((_TILE, _KP), 1)
    onehot = jnp.where(col == slot_i, 1.0, 0.0)
    out_ref[...] += jax.lax.dot_general(
        onehot, aug_ref[...], (((0,), (0,)), ((), ())),
        preferred_element_type=jnp.float32, precision=_HI)

    sm_ref[0, 2] = (base + jnp.sum(sel)).astype(jnp.int32)
    sm_ref[0, 3] = (tie_base + jnp.sum(tie_f)).astype(jnp.int32)


def _sort_nms_kernel(aug_ref, augt_ref, out_ref, iou_ref):
    aug = aug_ref[...]
    augt = augt_ref[...]
    lane_b = _fiota((1, _BLK), 1)
    lane_k = _fiota((1, _KP), 1)

    real_r = lane_k < float(_K)
    s_row = jnp.where(real_r, augt[6:7, :], -1.0)
    i_row = jnp.where(real_r, augt[7:8, :], 30000.0 + lane_k)

    rank_parts = []
    for b in range(_NBLK):
        base = b * _BLK
        rc = _fiota((_BLK, 1), 0) + float(base)
        s_col = jnp.where(rc < float(_K), aug[base:base + _BLK, 6:7], -1.0)
        i_col = jnp.where(rc < float(_K), aug[base:base + _BLK, 7:8],
                          30000.0 + rc)
        gt = s_row > s_col
        tie = (s_row == s_col) & (i_row < i_col)
        rank_parts.append(jnp.sum(jnp.where(gt | tie, 1.0, 0.0),
                                  axis=1, keepdims=True))
    rank = jnp.concatenate(rank_parts, axis=0)

    c_kp = _fiota((_BLK, _KP), 1)
    cand = jnp.zeros((_KP, 8), jnp.float32)
    candt = jnp.zeros((8, _KP), jnp.float32)
    for b in range(_NBLK):
        base = b * _BLK
        perm = jnp.where(c_kp == rank[base:base + _BLK, :], 1.0, 0.0)
        cand = cand + jax.lax.dot_general(
            perm, aug[base:base + _BLK, :], (((0,), (0,)), ((), ())),
            preferred_element_type=jnp.float32, precision=_HI)
        candt = candt + jax.lax.dot_general(
            augt[:, base:base + _BLK], perm, (((1,), (0,)), ((), ())),
            preferred_element_type=jnp.float32, precision=_HI)

    cidx8 = _iiota((_KP, 8), 1)
    cand = jnp.where(cidx8 < 6, jnp.clip(cand, 0.0, _SCENE), cand)
    ridx8 = _iiota((8, _KP), 0)
    candt = jnp.where(ridx8 < 6, jnp.clip(candt, 0.0, _SCENE), candt)

    for b in range(_NBLK):
        base = b * _BLK
        blk = cand[base:base + _BLK, :]
        inter = jnp.ones((_BLK, _KP), jnp.float32)
        vol_c = jnp.ones((_BLK, 1), jnp.float32)
        vol_r = jnp.ones((1, _KP), jnp.float32)
        for d in range(3):
            lo_c, hi_c = blk[:, d:d + 1], blk[:, d + 3:d + 4]
            lo_r, hi_r = candt[d:d + 1, :], candt[d + 3:d + 4, :]
            inter = inter * jnp.clip(jnp.minimum(hi_c, hi_r)
                                     - jnp.maximum(lo_c, lo_r), 0.0, None)
            vol_c = vol_c * (hi_c - lo_c)
            vol_r = vol_r * (hi_r - lo_r)
        union = vol_c + vol_r - inter
        iou_ref[base:base + _BLK, :] = inter / jnp.maximum(union, 1e-6)

    score_row = candt[6:7, :]
    keep = jnp.where(score_row > 0.0, 1.0, 0.0)

    for b in range(_NBLK):
        base = b * _BLK
        kb0 = keep[:, base:base + _BLK]
        tile = jnp.where(
            iou_ref[base:base + _BLK, base:base + _BLK] > _THR, 1.0, 0.0)

        def inner(i, kb):
            i_f = i.astype(jnp.float32)
            moh = jnp.where(lane_b == i_f, kb, 0.0)
            row = jax.lax.dot_general(
                moh, tile, (((1,), (0,)), ((), ())),
                preferred_element_type=jnp.float32)
            sup = (row > 0.5) & (lane_b > i_f)
            return jnp.where(sup, 0.0, kb)

        kb = jax.lax.fori_loop(0, _BLK, inner, kb0)

        pieces = []
        if base > 0:
            pieces.append(keep[:, :base])
        pieces.append(kb)
        if base + _BLK < _KP:
            pieces.append(keep[:, base + _BLK:])
        keep = jnp.concatenate(pieces, axis=1)

        if base + _BLK < _KP:
            panel = jnp.where(iou_ref[base:base + _BLK, :] > _THR, 1.0, 0.0)
            contrib = jax.lax.dot_general(
                kb, panel, (((1,), (0,)), ((), ())),
                preferred_element_type=jnp.float32)
            supp = (contrib > 0.5) & (lane_k >= float(base + _BLK))
            keep = jnp.where(supp, 0.0, keep)

    real = jnp.where(lane_k < float(_K), 1.0, 0.0)
    notk = real * (1.0 - keep)
    nk = jnp.sum(keep, axis=1, keepdims=True)

    csk_parts, csnk_parts = [], []
    r_kp = _fiota((_KP, _BLK), 0)
    c_blk = _fiota((_KP, _BLK), 1)
    for b in range(_NBLK):
        ut = jnp.where(r_kp <= c_blk + float(b * _BLK), 1.0, 0.0)
        csk_parts.append(jax.lax.dot_general(
            keep, ut, (((1,), (0,)), ((), ())),
            preferred_element_type=jnp.float32))
        csnk_parts.append(jax.lax.dot_general(
            notk, ut, (((1,), (0,)), ((), ())),
            preferred_element_type=jnp.float32))
    csk = jnp.concatenate(csk_parts, axis=1)
    csnk = jnp.concatenate(csnk_parts, axis=1)
    pos = jnp.where(keep > 0.5, csk - 1.0, nk + csnk - 1.0)
    pos = jnp.where(lane_k < float(_K), pos, 4.0 * _KP)

    r_blk = _fiota((_BLK, _KP), 0)
    acc = jnp.zeros((_KP, 8), jnp.float32)
    for b in range(_NBLK):
        base = b * _BLK
        sel = (c_kp == r_blk + float(base))
        pos_col = jnp.sum(jnp.where(sel, pos, 0.0),
                          axis=1, keepdims=True)
        keep_col = jnp.sum(jnp.where(sel, keep, 0.0),
                           axis=1, keepdims=True)
        perm = jnp.where(c_kp == pos_col, 1.0, 0.0)
        aug_blk = cand[base:base + _BLK, :]
        cidx = _iiota((_BLK, 8), 1)
        aug_blk = jnp.where(cidx == 7, keep_col, aug_blk)
        acc = acc + jax.lax.dot_general(
            perm, aug_blk, (((0,), (0,)), ((), ())),
            preferred_element_type=jnp.float32, precision=_HI)
    out_ref[...] = acc


def kernel(boxes, scores):
    pad = _NP - _N_RAW
    boxes_p = jnp.pad(boxes, ((0, pad), (0, 0)))
    scores_p = jnp.pad(scores, (0, pad), constant_values=-1.0)
    idx_col = jnp.arange(_NP, dtype=jnp.float32)[:, None]
    aug_all = jnp.concatenate(
        [boxes_p, scores_p[:, None], idx_col], axis=1)

    cand = pl.pallas_call(
        _select_kernel,
        grid=(_NP // _TILE,),
        in_specs=[
            pl.BlockSpec((_NP // 128, 128), lambda i: (0, 0)),
            pl.BlockSpec((_TILE, 1), lambda i: (i, 0)),
            pl.BlockSpec((_TILE, 8), lambda i: (i, 0)),
        ],
        out_specs=pl.BlockSpec((_KP, 8), lambda i: (0, 0)),
        out_shape=jax.ShapeDtypeStruct((_KP, 8), jnp.float32),
        scratch_shapes=[pltpu.SMEM((1, 4), jnp.int32)],
    )(scores_p.reshape(_NP // 128, 128), scores_p[:, None], aug_all)

    out = pl.pallas_call(
        _sort_nms_kernel,
        in_specs=[
            pl.BlockSpec((_KP, 8), lambda: (0, 0)),
            pl.BlockSpec((8, _KP), lambda: (0, 0)),
        ],
        out_specs=pl.BlockSpec((_KP, 8), lambda: (0, 0)),
        out_shape=jax.ShapeDtypeStruct((_KP, 8), jnp.float32),
        scratch_shapes=[pltpu.VMEM((_KP, _KP), jnp.float32)],
    )(cand, cand.T)

    proposals = out[:_K, :6]
    final_scores = jnp.where(out[:_K, 7] > 0.5, out[:_K, 6], -jnp.inf)
    return proposals, final_scores
